# baseline (speedup 1.0000x reference)
import dataclasses
import functools
import jax
import jax.numpy as jnp
from jax import lax
from jax.experimental import pallas as pl
from jax.experimental.pallas import tpu as pltpu
from jax.experimental.pallas import tpu_sc as plsc

N_NODES = 10000
N_EDGES = 320000
D = 128
DW = D // 2
NC = 2
NS = 16
NW = NC * NS
E_PER_W = N_EDGES // NW
CHUNK = 80
N_CHUNKS = E_PER_W // CHUNK
NBUF = 4
R_PER_W = N_NODES // NS
R_BLK = 125
N_RBLK = R_PER_W // R_BLK


def _dot_kernel(model_hbm, edge_hbm, out_hbm,
                packed_sh, packed_hbm, sidx_v, didx_v, rows0_v, rows1_v, out_v,
                pin_v, pout_v, gsem0, gsem1, osem):
  cid = lax.axis_index("c")
  sid = lax.axis_index("s")
  wid = sid * NC + cid
  ebase = wid * E_PER_W

  icp0 = pltpu.async_copy(edge_hbm.at[0, pl.ds(ebase, E_PER_W)], sidx_v,
                          gsem0.at[0])
  icp1 = pltpu.async_copy(edge_hbm.at[1, pl.ds(ebase, E_PER_W)], didx_v,
                          gsem1.at[0])

  @pl.loop(0, N_RBLK)
  def _pack(blk):
    row0 = sid * R_PER_W + blk * R_BLK
    pltpu.sync_copy(model_hbm.at[pl.ds(row0, R_BLK)], pin_v)

    @pl.loop(0, R_BLK)
    def _row(r):
      for k in range(D // 32):
        a = pin_v[r, pl.ds(32 * k, 16)]
        b = pin_v[r, pl.ds(32 * k + 16, 16)]
        p = plsc.pack(a, b, format=plsc.PackFormat.INTERLEAVED)
        pout_v[r, pl.ds(16 * k, 16)] = plsc.bitcast(p, jnp.int32)

    pltpu.sync_copy(pout_v, packed_sh.at[pl.ds(row0, R_BLK)])
    pltpu.sync_copy(pout_v, packed_hbm.at[cid, pl.ds(row0, R_BLK)])

  icp0.wait()
  icp1.wait()
  plsc.subcore_barrier()

  tables = (packed_sh, packed_hbm.at[cid])

  def issue_gather(chunk, b):
    table = tables[b % 2]
    s_idx = sidx_v.at[pl.ds(chunk * CHUNK, CHUNK)]
    d_idx = didx_v.at[pl.ds(chunk * CHUNK, CHUNK)]
    pltpu.async_copy(table.at[s_idx], rows0_v.at[b], gsem0.at[b])
    pltpu.async_copy(table.at[d_idx], rows1_v.at[b], gsem1.at[b])

  def wait_gather(b):
    table = tables[b % 2]
    s_idx = sidx_v.at[pl.ds(0, CHUNK)]
    d_idx = didx_v.at[pl.ds(0, CHUNK)]
    pltpu.make_async_copy(table.at[s_idx], rows0_v.at[b], gsem0.at[b]).wait()
    pltpu.make_async_copy(table.at[d_idx], rows1_v.at[b], gsem1.at[b]).wait()

  def out_store_wait(chunk, b):
    pltpu.make_async_copy(
        out_v.at[b], out_hbm.at[pl.ds(ebase + chunk * CHUNK, CHUNK)],
        osem.at[b]).wait()

  def compute(chunk, b):
    @pl.loop(0, CHUNK // 16)
    def _grp(g):
      outv = jnp.zeros((16,), jnp.float32)
      for j in range(16):
        e = g * 16 + j
        prods = []
        for k in range(D // 32):
          s = plsc.bitcast(rows0_v[b, e, pl.ds(16 * k, 16)], jnp.bfloat16)
          d = plsc.bitcast(rows1_v[b, e, pl.ds(16 * k, 16)], jnp.bfloat16)
          prods.append(s * d)
        acc_bf = (prods[0] + prods[1]) + (prods[2] + prods[3])
        u0, u1 = plsc.unpack(acc_bf, format=plsc.PackFormat.INTERLEAVED)
        mask = lax.iota(jnp.int32, 16) == j
        outv = jnp.where(mask, jnp.sum(u0 + u1), outv)
      out_v[b, pl.ds(g * 16, 16)] = outv

  issue_gather(0, 0)
  issue_gather(1, 1)
  issue_gather(2, 2)

  @pl.loop(0, N_CHUNKS - 1, step=NBUF)
  def _ring(c):
    for b in range(NBUF):
      chunk = c + b
      wait_gather(b)

      @pl.when(chunk + (NBUF - 1) <= N_CHUNKS - 1)
      def _():
        issue_gather(chunk + (NBUF - 1), (b + NBUF - 1) % NBUF)

      @pl.when(chunk >= NBUF)
      def _():
        out_store_wait(chunk - NBUF, b)

      compute(chunk, b)
      pltpu.async_copy(
          out_v.at[b], out_hbm.at[pl.ds(ebase + chunk * CHUNK, CHUNK)],
          osem.at[b])

  last = N_CHUNKS - 1
  wait_gather(0)
  out_store_wait(last - NBUF, 0)
  compute(last, 0)
  pltpu.sync_copy(out_v.at[0],
                  out_hbm.at[pl.ds(ebase + last * CHUNK, CHUNK)])
  out_store_wait(last - 3, 1)
  out_store_wait(last - 2, 2)
  out_store_wait(last - 1, 3)


@jax.jit
def kernel(model, edge_index):
  edge_index = edge_index.astype(jnp.int32)
  mesh = plsc.VectorSubcoreMesh(core_axis_name="c", subcore_axis_name="s")
  cp = pltpu.CompilerParams()
  if "needs_layout_passes" in pltpu.CompilerParams.__dataclass_fields__:
    cp = dataclasses.replace(cp, needs_layout_passes=False)
  cp = dataclasses.replace(cp, use_tc_tiling_on_sc=False)
  k = pl.kernel(
      _dot_kernel,
      out_type=jax.ShapeDtypeStruct((N_EDGES,), jnp.float32),
      mesh=mesh,
      scratch_types=[
          pltpu.VMEM_SHARED((N_NODES, DW), jnp.int32),
          pltpu.HBM((NC, N_NODES, DW), jnp.int32),
          pltpu.VMEM((E_PER_W,), jnp.int32),
          pltpu.VMEM((E_PER_W,), jnp.int32),
          pltpu.VMEM((NBUF, CHUNK, DW), jnp.int32),
          pltpu.VMEM((NBUF, CHUNK, DW), jnp.int32),
          pltpu.VMEM((NBUF, CHUNK), jnp.float32),
          pltpu.VMEM((R_BLK, D), jnp.float32),
          pltpu.VMEM((R_BLK, DW), jnp.int32),
          pltpu.SemaphoreType.DMA((NBUF,)),
          pltpu.SemaphoreType.DMA((NBUF,)),
          pltpu.SemaphoreType.DMA((NBUF,)),
      ],
      compiler_params=cp,
  )
  return k(model, edge_index)
